# baseline (speedup 1.0000x reference)
import functools

import jax
import jax.numpy as jnp
from jax import lax
from jax.experimental import pallas as pl
from jax.experimental.pallas import tpu as pltpu
from jax.experimental.pallas import tpu_sc as plsc

NC = 2
NS = 16
NW = NC * NS

_CR = 160
_CNCH = 1000000 // _CR


def _make_compact(V, D):
    assert V % _CR == 0 and D == 32
    mesh = plsc.VectorSubcoreMesh(core_axis_name="c", subcore_axis_name="s")

    @functools.partial(
        pl.kernel,
        mesh=mesh,
        out_type=jax.ShapeDtypeStruct((V * D // 128, 128), jnp.float32),
        scratch_types=[
            pltpu.VMEM((2, _CR, D), jnp.float32),
            pltpu.VMEM((2, _CR * D // 128, 128), jnp.float32),
            pltpu.SemaphoreType.DMA((2,)),
            pltpu.SemaphoreType.DMA((2,)),
        ],
        compiler_params=pltpu.CompilerParams(use_tc_tiling_on_sc=True),
    )
    def k(w_hbm, out_hbm, vin, vout, isem, osem):
        wid = lax.axis_index("s") * NC + lax.axis_index("c")
        n_r = _CR * D // 128

        def chunk_of(i):
            return wid + NW * i

        def start_in(i, buf):
            c = chunk_of(i)

            @pl.when(c < _CNCH)
            def _():
                pltpu.async_copy(
                    w_hbm.at[pl.ds(c * _CR, _CR), pl.ds(0, D)],
                    vin.at[buf], isem.at[buf],
                )

        def wait_in(i, buf):
            c = chunk_of(i)

            @pl.when(c < _CNCH)
            def _():
                pltpu.make_async_copy(
                    w_hbm.at[pl.ds(c * _CR, _CR), pl.ds(0, D)],
                    vin.at[buf], isem.at[buf],
                ).wait()

        def start_out(i, buf):
            c = chunk_of(i)

            @pl.when(c < _CNCH)
            def _():
                pltpu.async_copy(
                    vout.at[buf],
                    out_hbm.at[pl.ds(c * n_r, n_r), pl.ds(0, 128)],
                    osem.at[buf],
                )

        def wait_out(i, buf):
            c = chunk_of(i)

            @pl.when(c < _CNCH)
            def _():
                pltpu.make_async_copy(
                    vout.at[buf],
                    out_hbm.at[pl.ds(c * n_r, n_r), pl.ds(0, 128)],
                    osem.at[buf],
                ).wait()

        def repack(i, buf):
            c = chunk_of(i)

            @pl.when(c < _CNCH)
            def _():
                def body(k4, carry):
                    for u in range(2):
                        k = k4 * 2 + u
                        for j in range(4):
                            for h in range(2):
                                v = vin[buf, k * 4 + j, pl.ds(h * 16, 16)]
                                vout[buf, k, pl.ds(j * 32 + h * 16, 16)] = v
                    return carry

                lax.fori_loop(0, n_r // 2, body, 0)

        n_iter = -(-_CNCH // NW)

        def visit(i, buf, first, issue_next):
            wait_in(i, buf)
            if not first:
                wait_out(i, buf)
            repack(i, buf)
            if issue_next:
                start_in(i + 2, buf)
            start_out(i, buf)

        start_in(0, 0)
        start_in(1, 1)
        visit(0, 0, True, True)
        visit(1, 1, True, True)

        def body(i2, carry):
            for p in range(2):
                visit(i2 * 2 + p, p, False, True)
            return carry

        lax.fori_loop(1, (n_iter - 2) // 2, body, 0)

        visit(n_iter - 2, 0, False, True)
        visit(n_iter - 1, 1, False, True)
        wait_out(n_iter - 2, 0)
        wait_out(n_iter - 1, 1)

    return k


def _make_lookup(V, T, B, D):
    assert D == 32 and B % (NW * 128) == 0
    b_per_w = B // NW
    n_j = b_per_w // 128
    bq_per_w = b_per_w // 128

    mesh = plsc.VectorSubcoreMesh(core_axis_name="c", subcore_axis_name="s")

    @functools.partial(
        pl.kernel,
        mesh=mesh,
        out_type=jax.ShapeDtypeStruct((T, 4, B // 128, 8, 128), jnp.float32),
        scratch_types=[
            pltpu.VMEM((T, b_per_w), jnp.int32),
            pltpu.VMEM((2, n_j, 128, D), jnp.float32),
            pltpu.VMEM((2, 4, bq_per_w, 8, 129), jnp.float32),
            pltpu.SemaphoreType.DMA((2,)),
            pltpu.SemaphoreType.DMA((2,)),
        ],
        compiler_params=pltpu.CompilerParams(use_tc_tiling_on_sc=False, needs_layout_passes=False),
    )
    def k(table_hbm, idxT_hbm, out_hbm, idx_v, rows, blk, gsem, ssem):
        wid = lax.axis_index("s") * NC + lax.axis_index("c")
        b0 = wid * b_per_w
        bq0 = wid * bq_per_w

        pltpu.sync_copy(idxT_hbm.at[:, pl.ds(b0, b_per_w)], idx_v)

        cvec = lax.iota(jnp.int32, 16)
        cq0 = lax.shift_right_logical(cvec, 3)
        cq1 = cq0 + 2
        cr_v = lax.bitwise_and(cvec, 7)

        def start_gathers(t, buf):
            for j in range(n_j):
                pltpu.async_copy(
                    table_hbm.at[idx_v.at[t, pl.ds(j * 128, 128)]],
                    rows.at[buf, j], gsem.at[buf],
                )

        def wait_gathers(t, buf):
            for j in range(n_j):
                pltpu.make_async_copy(
                    table_hbm.at[idx_v.at[t, pl.ds(j * 128, 128)]],
                    rows.at[buf, j], gsem.at[buf],
                ).wait()

        def start_stores(t, buf):
            for cq in range(4):
                pltpu.async_copy(
                    blk.at[buf, cq, pl.ds(0, bq_per_w), pl.ds(0, 8),
                           pl.ds(0, 128)],
                    out_hbm.at[t, cq, pl.ds(bq0, bq_per_w)],
                    ssem.at[buf],
                )

        def wait_stores(t, buf):
            for cq in range(4):
                pltpu.make_async_copy(
                    blk.at[buf, cq, pl.ds(0, bq_per_w), pl.ds(0, 8),
                           pl.ds(0, 128)],
                    out_hbm.at[t, cq, pl.ds(bq0, bq_per_w)],
                    ssem.at[buf],
                ).wait()

        def transpose_block(buf):
            rows_ref = rows.at[buf]
            blk_ref = blk.at[buf]
            for j in range(n_j):
                j_v = jnp.full((16,), j, jnp.int32)

                def body(br4, carry, *, j=j, j_v=j_v):
                    for u in range(4):
                        br = br4 * 4 + u
                        br_v = jnp.full((16,), br, jnp.int32)
                        v0 = rows_ref[j, br, pl.ds(0, 16)]
                        v1 = rows_ref[j, br, pl.ds(16, 16)]
                        plsc.store_scatter(blk_ref, [cq0, j_v, cr_v, br_v], v0)
                        plsc.store_scatter(blk_ref, [cq1, j_v, cr_v, br_v], v1)
                    return carry

                lax.fori_loop(0, 32, body, 0)

        def visit(t, buf, first, issue_next):
            wait_gathers(t, buf)
            if not first:
                wait_stores(t, buf)
            transpose_block(buf)
            if issue_next:
                start_gathers(t + 2, buf)
            start_stores(t, buf)

        start_gathers(0, 0)
        start_gathers(1, 1)
        visit(0, 0, True, True)
        visit(1, 1, True, True)

        def body(t2, carry):
            for p in range(2):
                visit(t2 * 2 + p, p, False, True)
            return carry

        lax.fori_loop(1, (T - 2) // 2, body, 0)

        visit(T - 2, 0, False, False)
        visit(T - 1, 1, False, False)
        wait_stores(T - 2, 0)
        wait_stores(T - 1, 1)

    return k


def kernel(inputs, embedding_weight):
    Bdim, T = inputs.shape
    V, D = embedding_weight.shape
    idxT = inputs.T.astype(jnp.int32)
    compact = _make_compact(V, D)
    packed = compact(embedding_weight).reshape(V, D)
    lookup = _make_lookup(V, T, Bdim, D)
    out5 = lookup(packed, idxT)
    return out5.transpose(2, 4, 0, 1, 3).reshape(Bdim, T, D)


# speedup vs baseline: 1.0500x; 1.0500x over previous
import functools

import jax
import jax.numpy as jnp
from jax import lax
from jax.experimental import pallas as pl
from jax.experimental.pallas import tpu as pltpu
from jax.experimental.pallas import tpu_sc as plsc

NC = 2
NS = 16
NW = NC * NS

_CR = 320
_CNCH = 1000000 // _CR


def _make_compact(V, D):
    assert V % _CR == 0 and D == 32
    mesh = plsc.VectorSubcoreMesh(core_axis_name="c", subcore_axis_name="s")

    @functools.partial(
        pl.kernel,
        mesh=mesh,
        out_type=jax.ShapeDtypeStruct((V * D // 128, 128), jnp.float32),
        scratch_types=[
            pltpu.VMEM((2, _CR, D), jnp.float32),
            pltpu.VMEM((2, _CR * D // 128, 128), jnp.float32),
            pltpu.SemaphoreType.DMA((2,)),
            pltpu.SemaphoreType.DMA((2,)),
        ],
        compiler_params=pltpu.CompilerParams(use_tc_tiling_on_sc=True),
    )
    def k(w_hbm, out_hbm, vin, vout, isem, osem):
        wid = lax.axis_index("s") * NC + lax.axis_index("c")
        n_r = _CR * D // 128

        def chunk_of(i):
            return wid + NW * i

        def start_in(i, buf):
            c = chunk_of(i)

            @pl.when(c < _CNCH)
            def _():
                pltpu.async_copy(
                    w_hbm.at[pl.ds(c * _CR, _CR), pl.ds(0, D)],
                    vin.at[buf], isem.at[buf],
                )

        def wait_in(i, buf):
            c = chunk_of(i)

            @pl.when(c < _CNCH)
            def _():
                pltpu.make_async_copy(
                    w_hbm.at[pl.ds(c * _CR, _CR), pl.ds(0, D)],
                    vin.at[buf], isem.at[buf],
                ).wait()

        def start_out(i, buf):
            c = chunk_of(i)

            @pl.when(c < _CNCH)
            def _():
                pltpu.async_copy(
                    vout.at[buf],
                    out_hbm.at[pl.ds(c * n_r, n_r), pl.ds(0, 128)],
                    osem.at[buf],
                )

        def wait_out(i, buf):
            c = chunk_of(i)

            @pl.when(c < _CNCH)
            def _():
                pltpu.make_async_copy(
                    vout.at[buf],
                    out_hbm.at[pl.ds(c * n_r, n_r), pl.ds(0, 128)],
                    osem.at[buf],
                ).wait()

        def repack(i, buf):
            c = chunk_of(i)

            @pl.when(c < _CNCH)
            def _():
                def body(k8, carry):
                    for u in range(8):
                        k = k8 * 8 + u
                        for j in range(4):
                            for h in range(2):
                                v = vin[buf, k * 4 + j, pl.ds(h * 16, 16)]
                                vout[buf, k, pl.ds(j * 32 + h * 16, 16)] = v
                    return carry

                lax.fori_loop(0, n_r // 8, body, 0)

        n_iter = -(-_CNCH // NW)

        def visit(i, buf, first, issue_next):
            wait_in(i, buf)
            if not first:
                wait_out(i, buf)
            repack(i, buf)
            if issue_next:
                start_in(i + 2, buf)
            start_out(i, buf)

        start_in(0, 0)
        start_in(1, 1)
        visit(0, 0, True, True)
        visit(1, 1, True, True)

        def body(i2, carry):
            for p in range(2):
                visit(i2 * 2 + p, p, False, True)
            return carry

        lax.fori_loop(1, (n_iter - 2) // 2, body, 0)

        visit(n_iter - 2, 0, False, True)
        visit(n_iter - 1, 1, False, True)
        wait_out(n_iter - 2, 0)
        wait_out(n_iter - 1, 1)

    return k


def _make_lookup(V, T, B, D):
    assert D == 32 and B % (NW * 128) == 0
    b_per_w = B // NW
    n_j = b_per_w // 128
    bq_per_w = b_per_w // 128

    mesh = plsc.VectorSubcoreMesh(core_axis_name="c", subcore_axis_name="s")

    @functools.partial(
        pl.kernel,
        mesh=mesh,
        out_type=jax.ShapeDtypeStruct((T, 4, B // 128, 8, 128), jnp.float32),
        scratch_types=[
            pltpu.VMEM((T, b_per_w), jnp.int32),
            pltpu.VMEM((2, n_j, 128, D), jnp.float32),
            pltpu.VMEM((2, 4, bq_per_w, 8, 129), jnp.float32),
            pltpu.SemaphoreType.DMA((2,)),
            pltpu.SemaphoreType.DMA((2,)),
        ],
        compiler_params=pltpu.CompilerParams(use_tc_tiling_on_sc=False, needs_layout_passes=False),
    )
    def k(table_hbm, idxT_hbm, out_hbm, idx_v, rows, blk, gsem, ssem):
        wid = lax.axis_index("s") * NC + lax.axis_index("c")
        b0 = wid * b_per_w
        bq0 = wid * bq_per_w

        pltpu.sync_copy(idxT_hbm.at[:, pl.ds(b0, b_per_w)], idx_v)

        cvec = lax.iota(jnp.int32, 16)
        cq0 = lax.shift_right_logical(cvec, 3)
        cq1 = cq0 + 2
        cr_v = lax.bitwise_and(cvec, 7)

        def start_gathers(t, buf):
            for j in range(n_j):
                pltpu.async_copy(
                    table_hbm.at[idx_v.at[t, pl.ds(j * 128, 128)]],
                    rows.at[buf, j], gsem.at[buf],
                )

        def wait_gathers(t, buf):
            for j in range(n_j):
                pltpu.make_async_copy(
                    table_hbm.at[idx_v.at[t, pl.ds(j * 128, 128)]],
                    rows.at[buf, j], gsem.at[buf],
                ).wait()

        def start_stores(t, buf):
            for cq in range(4):
                pltpu.async_copy(
                    blk.at[buf, cq, pl.ds(0, bq_per_w), pl.ds(0, 8),
                           pl.ds(0, 128)],
                    out_hbm.at[t, cq, pl.ds(bq0, bq_per_w)],
                    ssem.at[buf],
                )

        def wait_stores(t, buf):
            for cq in range(4):
                pltpu.make_async_copy(
                    blk.at[buf, cq, pl.ds(0, bq_per_w), pl.ds(0, 8),
                           pl.ds(0, 128)],
                    out_hbm.at[t, cq, pl.ds(bq0, bq_per_w)],
                    ssem.at[buf],
                ).wait()

        def transpose_block(buf):
            rows_ref = rows.at[buf]
            blk_ref = blk.at[buf]
            for j in range(n_j):
                j_v = jnp.full((16,), j, jnp.int32)

                def body(br4, carry, *, j=j, j_v=j_v):
                    for u in range(4):
                        br = br4 * 4 + u
                        br_v = jnp.full((16,), br, jnp.int32)
                        v0 = rows_ref[j, br, pl.ds(0, 16)]
                        v1 = rows_ref[j, br, pl.ds(16, 16)]
                        plsc.store_scatter(blk_ref, [cq0, j_v, cr_v, br_v], v0)
                        plsc.store_scatter(blk_ref, [cq1, j_v, cr_v, br_v], v1)
                    return carry

                lax.fori_loop(0, 32, body, 0)

        def visit(t, buf, first, issue_next):
            wait_gathers(t, buf)
            if not first:
                wait_stores(t, buf)
            transpose_block(buf)
            if issue_next:
                start_gathers(t + 2, buf)
            start_stores(t, buf)

        start_gathers(0, 0)
        start_gathers(1, 1)
        visit(0, 0, True, True)
        visit(1, 1, True, True)

        def body(t2, carry):
            for p in range(2):
                visit(t2 * 2 + p, p, False, True)
            return carry

        lax.fori_loop(1, (T - 2) // 2, body, 0)

        visit(T - 2, 0, False, False)
        visit(T - 1, 1, False, False)
        wait_stores(T - 2, 0)
        wait_stores(T - 1, 1)

    return k


def kernel(inputs, embedding_weight):
    Bdim, T = inputs.shape
    V, D = embedding_weight.shape
    idxT = inputs.T.astype(jnp.int32)
    compact = _make_compact(V, D)
    packed = compact(embedding_weight).reshape(V, D)
    lookup = _make_lookup(V, T, Bdim, D)
    out5 = lookup(packed, idxT)
    return out5.transpose(2, 4, 0, 1, 3).reshape(Bdim, T, D)


# speedup vs baseline: 1.1145x; 1.0614x over previous
import functools

import jax
import jax.numpy as jnp
from jax import lax
from jax.experimental import pallas as pl
from jax.experimental.pallas import tpu as pltpu
from jax.experimental.pallas import tpu_sc as plsc

NC = 2
NS = 16
NW = NC * NS

def _make_lookup(V, T, B, D):
    assert D == 32 and B % (NW * 128) == 0
    b_per_w = B // NW
    n_j = b_per_w // 128
    bq_per_w = b_per_w // 128

    mesh = plsc.VectorSubcoreMesh(core_axis_name="c", subcore_axis_name="s")

    @functools.partial(
        pl.kernel,
        mesh=mesh,
        out_type=jax.ShapeDtypeStruct((T, 4, B // 128, 8, 128), jnp.float32),
        scratch_types=[
            pltpu.VMEM((T, b_per_w), jnp.int32),
            pltpu.VMEM((2, n_j, 128, D), jnp.float32),
            pltpu.VMEM((2, 4, bq_per_w, 8, 129), jnp.float32),
            pltpu.SemaphoreType.DMA((2,)),
            pltpu.SemaphoreType.DMA((2,)),
        ],
        compiler_params=pltpu.CompilerParams(use_tc_tiling_on_sc=False, needs_layout_passes=False),
    )
    def k(table_hbm, idxT_hbm, out_hbm, idx_v, rows, blk, gsem, ssem):
        wid = lax.axis_index("s") * NC + lax.axis_index("c")
        b0 = wid * b_per_w
        bq0 = wid * bq_per_w

        pltpu.sync_copy(idxT_hbm.at[:, pl.ds(b0, b_per_w)], idx_v)

        cvec = lax.iota(jnp.int32, 16)
        cq0 = lax.shift_right_logical(cvec, 3)
        cq1 = cq0 + 2
        cr_v = lax.bitwise_and(cvec, 7)

        def start_gathers(t, buf):
            for j in range(n_j):
                pltpu.async_copy(
                    table_hbm.at[idx_v.at[t, pl.ds(j * 128, 128)]],
                    rows.at[buf, j], gsem.at[buf],
                )

        def wait_gathers(t, buf):
            for j in range(n_j):
                pltpu.make_async_copy(
                    table_hbm.at[idx_v.at[t, pl.ds(j * 128, 128)]],
                    rows.at[buf, j], gsem.at[buf],
                ).wait()

        def start_stores(t, buf):
            for cq in range(4):
                pltpu.async_copy(
                    blk.at[buf, cq, pl.ds(0, bq_per_w), pl.ds(0, 8),
                           pl.ds(0, 128)],
                    out_hbm.at[t, cq, pl.ds(bq0, bq_per_w)],
                    ssem.at[buf],
                )

        def wait_stores(t, buf):
            for cq in range(4):
                pltpu.make_async_copy(
                    blk.at[buf, cq, pl.ds(0, bq_per_w), pl.ds(0, 8),
                           pl.ds(0, 128)],
                    out_hbm.at[t, cq, pl.ds(bq0, bq_per_w)],
                    ssem.at[buf],
                ).wait()

        def transpose_block(buf):
            rows_ref = rows.at[buf]
            blk_ref = blk.at[buf]
            for j in range(n_j):
                j_v = jnp.full((16,), j, jnp.int32)

                def body(br4, carry, *, j=j, j_v=j_v):
                    for u in range(4):
                        br = br4 * 4 + u
                        br_v = jnp.full((16,), br, jnp.int32)
                        v0 = rows_ref[j, br, pl.ds(0, 16)]
                        v1 = rows_ref[j, br, pl.ds(16, 16)]
                        plsc.store_scatter(blk_ref, [cq0, j_v, cr_v, br_v], v0)
                        plsc.store_scatter(blk_ref, [cq1, j_v, cr_v, br_v], v1)
                    return carry

                lax.fori_loop(0, 32, body, 0)

        def visit(t, buf, first, issue_next):
            wait_gathers(t, buf)
            if not first:
                wait_stores(t, buf)
            transpose_block(buf)
            if issue_next:
                start_gathers(t + 2, buf)
            start_stores(t, buf)

        start_gathers(0, 0)
        start_gathers(1, 1)
        visit(0, 0, True, True)
        visit(1, 1, True, True)

        def body(t2, carry):
            for p in range(2):
                visit(t2 * 2 + p, p, False, True)
            return carry

        lax.fori_loop(1, (T - 2) // 2, body, 0)

        visit(T - 2, 0, False, False)
        visit(T - 1, 1, False, False)
        wait_stores(T - 2, 0)
        wait_stores(T - 1, 1)

    return k


def kernel(inputs, embedding_weight):
    Bdim, T = inputs.shape
    V, D = embedding_weight.shape
    idxT = inputs.T.astype(jnp.int32)
    lookup = _make_lookup(V, T, Bdim, D)
    out5 = lookup(embedding_weight, idxT)
    return out5.transpose(2, 4, 0, 1, 3).reshape(Bdim, T, D)
